# Initial kernel scaffold; baseline (speedup 1.0000x reference)
import jax
import jax.numpy as jnp
from jax.experimental import pallas as pl


def kernel(x, embed, W1, b1, W2, b2):
    raise NotImplementedError("write your pallas kernel here")


# speedup vs baseline: 12.2548x; 12.2548x over previous
import functools

import jax
import jax.numpy as jnp
from jax import lax
from jax.experimental import pallas as pl
from jax.experimental.pallas import tpu as pltpu
from jax.experimental.pallas import tpu_sc as plsc

_NC = 2
_NS = 16
_NW = _NC * _NS
_LANES = 16


def _make_sc_sum(B, L, V, E):
    bpw = B // _NW
    assert B % _NW == 0 and E % _LANES == 0 and bpw % _LANES == 0

    mesh = plsc.VectorSubcoreMesh(core_axis_name="c", subcore_axis_name="s")

    @functools.partial(
        pl.kernel,
        out_type=(
            jax.ShapeDtypeStruct((B, E), jnp.float32),
            jax.ShapeDtypeStruct((B,), jnp.float32),
        ),
        mesh=mesh,
        scratch_types=[
            pltpu.VMEM((L, bpw), jnp.int32),
            pltpu.VMEM((bpw, E), jnp.float32),
            pltpu.VMEM((bpw,), jnp.float32),
            pltpu.SemaphoreType.DMA,
        ],
    )
    def sc_sum(xT_hbm, embed_hbm, sum_hbm, cnt_hbm, idx_v, acc_v, cnt_v, sem):
        wid = lax.axis_index("s") * _NC + lax.axis_index("c")
        base = wid * bpw

        pltpu.sync_copy(xT_hbm.at[:, pl.ds(base, bpw)], idx_v)

        pltpu.async_copy(embed_hbm.at[idx_v.at[0]], acc_v, sem).wait()

        chunk = 12
        j = 1
        while j < L:
            hi = min(L, j + chunk)
            descs = [
                pltpu.async_copy(embed_hbm.at[idx_v.at[jj]], acc_v, sem, add=True)
                for jj in range(j, hi)
            ]
            for d in descs:
                d.wait()
            j = hi

        nchunks = bpw // _LANES
        ones = jnp.ones((_LANES,), jnp.float32)
        zeros = jnp.zeros((_LANES,), jnp.float32)

        def cbody(j, carry):
            out = []
            for c in range(nchunks):
                v = idx_v[j, pl.ds(c * _LANES, _LANES)]
                out.append(carry[c] + jnp.where(v != 0, ones, zeros))
            return tuple(out)

        cnts = lax.fori_loop(0, L, cbody, tuple(zeros for _ in range(nchunks)))
        for c in range(nchunks):
            cnt_v[pl.ds(c * _LANES, _LANES)] = cnts[c]

        pltpu.sync_copy(acc_v, sum_hbm.at[pl.ds(base, bpw), :])
        pltpu.sync_copy(cnt_v, cnt_hbm.at[pl.ds(base, bpw)])

    return sc_sum


def _make_mlp(B, E, H, NP):
    BK = 512

    def body(sum_ref, cnt_ref, w1_ref, b1_ref, w2_ref, b2_ref, out_ref):
        s = sum_ref[...]
        c = cnt_ref[...]
        avg = s * (1.0 / jnp.maximum(c, 1e-6))
        h = lax.dot_general(
            avg, w1_ref[...], (((1,), (0,)), ((), ())),
            precision=lax.Precision.HIGHEST,
            preferred_element_type=jnp.float32,
        ) + b1_ref[...]
        h = jnp.maximum(h, 0.0)
        out_ref[...] = lax.dot_general(
            h, w2_ref[...], (((1,), (0,)), ((), ())),
            precision=lax.Precision.HIGHEST,
            preferred_element_type=jnp.float32,
        ) + b2_ref[...]

    return pl.pallas_call(
        body,
        grid=(B // BK,),
        in_specs=[
            pl.BlockSpec((BK, E), lambda i: (i, 0)),
            pl.BlockSpec((BK, 1), lambda i: (i, 0)),
            pl.BlockSpec((E, H), lambda i: (0, 0)),
            pl.BlockSpec((1, H), lambda i: (0, 0)),
            pl.BlockSpec((H, NP), lambda i: (0, 0)),
            pl.BlockSpec((1, NP), lambda i: (0, 0)),
        ],
        out_specs=pl.BlockSpec((BK, NP), lambda i: (i, 0)),
        out_shape=jax.ShapeDtypeStruct((B, NP), jnp.float32),
    )


def kernel(x, embed, W1, b1, W2, b2):
    B, L = x.shape
    V, E = embed.shape
    H = W1.shape[1]
    N = W2.shape[1]
    NP = ((N + 127) // 128) * 128

    xT = jnp.transpose(x).astype(jnp.int32)
    summed, cnt = _make_sc_sum(B, L, V, E)(xT, embed)

    W2p = jnp.pad(W2, ((0, 0), (0, NP - N)))
    b2p = jnp.pad(b2, (0, NP - N))
    out = _make_mlp(B, E, H, NP)(
        summed, cnt.reshape(B, 1), W1, b1.reshape(1, H), W2p, b2p.reshape(1, NP)
    )
    return out[:, :N]


# speedup vs baseline: 12.5195x; 1.0216x over previous
import functools

import jax
import jax.numpy as jnp
from jax import lax
from jax.experimental import pallas as pl
from jax.experimental.pallas import tpu as pltpu
from jax.experimental.pallas import tpu_sc as plsc

_NC = 2
_NS = 16
_NW = _NC * _NS
_LANES = 16


def _make_sc_sum(B, L, V, E):
    bpw = B // _NW
    assert B % _NW == 0 and E % _LANES == 0 and bpw % _LANES == 0

    mesh = plsc.VectorSubcoreMesh(core_axis_name="c", subcore_axis_name="s")

    @functools.partial(
        pl.kernel,
        out_type=(
            jax.ShapeDtypeStruct((B, E), jnp.float32),
            jax.ShapeDtypeStruct((B,), jnp.float32),
        ),
        mesh=mesh,
        scratch_types=[
            pltpu.VMEM((L, bpw), jnp.int32),
            pltpu.VMEM((bpw, E), jnp.float32),
            pltpu.VMEM((bpw,), jnp.float32),
            pltpu.SemaphoreType.DMA,
        ],
    )
    def sc_sum(xT_hbm, embed_hbm, sum_hbm, cnt_hbm, idx_v, acc_v, cnt_v, sem):
        wid = lax.axis_index("s") * _NC + lax.axis_index("c")
        base = wid * bpw

        pltpu.sync_copy(xT_hbm.at[:, pl.ds(base, bpw)], idx_v)

        pltpu.async_copy(embed_hbm.at[idx_v.at[0]], acc_v, sem).wait()

        descs = [
            pltpu.async_copy(embed_hbm.at[idx_v.at[jj]], acc_v, sem, add=True)
            for jj in range(1, L)
        ]

        nchunks = bpw // _LANES
        ones = jnp.ones((_LANES,), jnp.float32)
        zeros = jnp.zeros((_LANES,), jnp.float32)

        def cbody(j, carry):
            out = []
            for c in range(nchunks):
                v = idx_v[j, pl.ds(c * _LANES, _LANES)]
                out.append(carry[c] + jnp.where(v != 0, ones, zeros))
            return tuple(out)

        cnts = lax.fori_loop(0, L, cbody, tuple(zeros for _ in range(nchunks)))
        for c in range(nchunks):
            cnt_v[pl.ds(c * _LANES, _LANES)] = cnts[c]

        for d in descs:
            d.wait()

        pltpu.sync_copy(acc_v, sum_hbm.at[pl.ds(base, bpw), :])
        pltpu.sync_copy(cnt_v, cnt_hbm.at[pl.ds(base, bpw)])

    return sc_sum


def _make_mlp(B, E, H, NP):
    BK = 512

    def body(sum_ref, cnt_ref, w1_ref, b1_ref, w2_ref, b2_ref, out_ref):
        s = sum_ref[...]
        c = cnt_ref[...]
        avg = s * (1.0 / jnp.maximum(c, 1e-6))
        h = lax.dot_general(
            avg, w1_ref[...], (((1,), (0,)), ((), ())),
            precision=lax.Precision.HIGHEST,
            preferred_element_type=jnp.float32,
        ) + b1_ref[...]
        h = jnp.maximum(h, 0.0)
        out_ref[...] = lax.dot_general(
            h, w2_ref[...], (((1,), (0,)), ((), ())),
            precision=lax.Precision.HIGHEST,
            preferred_element_type=jnp.float32,
        ) + b2_ref[...]

    return pl.pallas_call(
        body,
        grid=(B // BK,),
        in_specs=[
            pl.BlockSpec((BK, E), lambda i: (i, 0)),
            pl.BlockSpec((BK, 1), lambda i: (i, 0)),
            pl.BlockSpec((E, H), lambda i: (0, 0)),
            pl.BlockSpec((1, H), lambda i: (0, 0)),
            pl.BlockSpec((H, NP), lambda i: (0, 0)),
            pl.BlockSpec((1, NP), lambda i: (0, 0)),
        ],
        out_specs=pl.BlockSpec((BK, NP), lambda i: (i, 0)),
        out_shape=jax.ShapeDtypeStruct((B, NP), jnp.float32),
    )


def kernel(x, embed, W1, b1, W2, b2):
    B, L = x.shape
    V, E = embed.shape
    H = W1.shape[1]
    N = W2.shape[1]
    NP = ((N + 127) // 128) * 128

    xT = jnp.transpose(x).astype(jnp.int32)
    summed, cnt = _make_sc_sum(B, L, V, E)(xT, embed)

    W2p = jnp.pad(W2, ((0, 0), (0, NP - N)))
    b2p = jnp.pad(b2, (0, NP - N))
    out = _make_mlp(B, E, H, NP)(
        summed, cnt.reshape(B, 1), W1, b1.reshape(1, H), W2p, b2p.reshape(1, NP)
    )
    return out[:, :N]


# speedup vs baseline: 12.7184x; 1.0159x over previous
import functools

import jax
import jax.numpy as jnp
from jax import lax
from jax.experimental import pallas as pl
from jax.experimental.pallas import tpu as pltpu
from jax.experimental.pallas import tpu_sc as plsc

_NC = 2
_NS = 16
_NW = _NC * _NS
_LANES = 16


def _make_sc_sum(B, L, V, E):
    bpw = B // _NW
    assert B % _NW == 0 and E % _LANES == 0 and bpw % _LANES == 0

    mesh = plsc.VectorSubcoreMesh(core_axis_name="c", subcore_axis_name="s")

    @functools.partial(
        pl.kernel,
        out_type=(
            jax.ShapeDtypeStruct((B, E), jnp.float32),
            jax.ShapeDtypeStruct((B,), jnp.float32),
        ),
        mesh=mesh,
        scratch_types=[
            pltpu.VMEM((L, bpw), jnp.int32),
            pltpu.VMEM((bpw, E), jnp.float32),
            pltpu.VMEM((bpw,), jnp.float32),
            pltpu.SemaphoreType.DMA,
        ],
    )
    def sc_sum(xT_hbm, embed_hbm, sum_hbm, cnt_hbm, idx_v, acc_v, cnt_v, sem):
        wid = lax.axis_index("s") * _NC + lax.axis_index("c")
        base = wid * bpw

        pltpu.sync_copy(xT_hbm.at[:, pl.ds(base, bpw)], idx_v)

        pltpu.async_copy(embed_hbm.at[idx_v.at[0]], acc_v, sem).wait()

        descs = [
            pltpu.async_copy(embed_hbm.at[idx_v.at[jj]], acc_v, sem, add=True)
            for jj in range(1, L)
        ]

        nchunks = bpw // _LANES
        ones = jnp.ones((_LANES,), jnp.float32)
        zeros = jnp.zeros((_LANES,), jnp.float32)

        def cbody(j, carry):
            out = []
            for c in range(nchunks):
                v = idx_v[j, pl.ds(c * _LANES, _LANES)]
                out.append(carry[c] + jnp.where(v != 0, ones, zeros))
            return tuple(out)

        cnts = lax.fori_loop(0, L, cbody, tuple(zeros for _ in range(nchunks)))
        for c in range(nchunks):
            cnt_v[pl.ds(c * _LANES, _LANES)] = cnts[c]

        for d in descs:
            d.wait()

        pltpu.sync_copy(acc_v, sum_hbm.at[pl.ds(base, bpw), :])
        pltpu.sync_copy(cnt_v, cnt_hbm.at[pl.ds(base, bpw)])

    return sc_sum


def _make_mlp(B, E, H, N):
    BK = 1024

    def body(sum_ref, cnt_ref, w1_ref, b1_ref, w2_ref, b2_ref, out_ref):
        s = sum_ref[...]
        c = cnt_ref[...]
        avg = s * (1.0 / jnp.maximum(c, 1e-6))
        h = lax.dot_general(
            avg, w1_ref[...], (((1,), (0,)), ((), ())),
            precision=lax.Precision.HIGHEST,
            preferred_element_type=jnp.float32,
        ) + b1_ref[...]
        h = jnp.maximum(h, 0.0)
        out_ref[...] = lax.dot_general(
            h, w2_ref[...], (((1,), (0,)), ((), ())),
            precision=lax.Precision.HIGHEST,
            preferred_element_type=jnp.float32,
        ) + b2_ref[...]

    return pl.pallas_call(
        body,
        grid=(B // BK,),
        in_specs=[
            pl.BlockSpec((BK, E), lambda i: (i, 0)),
            pl.BlockSpec((BK, 1), lambda i: (i, 0)),
            pl.BlockSpec((E, H), lambda i: (0, 0)),
            pl.BlockSpec((1, H), lambda i: (0, 0)),
            pl.BlockSpec((H, N), lambda i: (0, 0)),
            pl.BlockSpec((1, N), lambda i: (0, 0)),
        ],
        out_specs=pl.BlockSpec((BK, N), lambda i: (i, 0)),
        out_shape=jax.ShapeDtypeStruct((B, N), jnp.float32),
    )


def kernel(x, embed, W1, b1, W2, b2):
    B, L = x.shape
    V, E = embed.shape
    H = W1.shape[1]
    N = W2.shape[1]

    xT = jnp.transpose(x).astype(jnp.int32)
    summed, cnt = _make_sc_sum(B, L, V, E)(xT, embed)

    return _make_mlp(B, E, H, N)(
        summed, cnt.reshape(B, 1), W1, b1.reshape(1, H), W2, b2.reshape(1, N)
    )
